# baseline (speedup 1.0000x reference)
import functools

import jax
import jax.numpy as jnp
from jax import lax
from jax.experimental import pallas as pl
from jax.experimental.pallas import tpu as pltpu
from jax.experimental.pallas import tpu_sc as plsc

N = 10000
NR = 10240
E = 320000
NSC = 2
NTILE = 16
CHUNK = 128

NCH16 = 160
NCH32 = 80
T16 = NCH16 // 2
T32 = NCH32 // 2
EPT16 = NCH16 * CHUNK
EPT32 = NCH32 * CHUNK
EPAD = EPT16 * NTILE

TROWS = NR // NTILE

BN = 2048
GRID = NR // BN


@functools.lru_cache(maxsize=None)
def _make_agg(split):
  nit = T16 if split else T32
  mesh = plsc.VectorSubcoreMesh(core_axis_name="c", subcore_axis_name="s")

  @functools.partial(
      pl.kernel,
      out_type=jax.ShapeDtypeStruct((NSC, NR, 128), jnp.float32),
      scratch_types=[
          pltpu.VMEM((2, 2, CHUNK), jnp.int32),
          pltpu.VMEM((2, 2, CHUNK), jnp.int32),
          pltpu.VMEM((2, CHUNK, 128), jnp.float32),
          pltpu.VMEM_SHARED((NR, 128), jnp.float32),
      ] + [pltpu.SemaphoreType.DMA] * 4,
      mesh=mesh,
  )
  def agg(table, srcg, dstg, zrows, out, srcb, dstb, rowb, acc,
          gsem0, gsem1, ssem0, ssem1):
    c = lax.axis_index("c")
    s = lax.axis_index("s")
    rbase = pl.multiple_of(s * TROWS, TROWS)

    if split:
      src_it = lambda t: srcg.at[c, s, t]
      dst_it = lambda t: dstg.at[s, t]
    else:
      w = c * NTILE + s
      src_it = lambda t: srcg.at[w, t]
      dst_it = lambda t: dstg.at[w, t]

    pltpu.sync_copy(src_it(0), srcb.at[0])
    pltpu.sync_copy(dst_it(0), dstb.at[0])
    pltpu.sync_copy(zrows, acc.at[pl.ds(rbase, TROWS)])
    plsc.subcore_barrier()

    def body(t, carry):
      p = lax.rem(t, 2)
      g0 = pltpu.async_copy(table.at[srcb.at[p, 0]], rowb.at[0], gsem0)
      g1 = pltpu.async_copy(table.at[srcb.at[p, 1]], rowb.at[1], gsem1)

      @pl.when(t + 1 < nit)
      def _():
        pltpu.sync_copy(src_it(t + 1), srcb.at[1 - p])
        pltpu.sync_copy(dst_it(t + 1), dstb.at[1 - p])

      g0.wait()
      s0 = pltpu.async_copy(rowb.at[0], acc.at[dstb.at[p, 0]], ssem0, add=True)
      g1.wait()
      s1 = pltpu.async_copy(rowb.at[1], acc.at[dstb.at[p, 1]], ssem1, add=True)
      s0.wait()
      s1.wait()
      return carry

    lax.fori_loop(0, nit, body, 0)
    plsc.subcore_barrier()
    pltpu.sync_copy(acc.at[pl.ds(rbase, TROWS)], out.at[c, pl.ds(rbase, TROWS)])

  return agg


@functools.lru_cache(maxsize=None)
def _make_deg():
  mesh = plsc.VectorSubcoreMesh(core_axis_name="c", subcore_axis_name="s")

  @functools.partial(
      pl.kernel,
      out_type=jax.ShapeDtypeStruct((NSC, NR, 128), jnp.float32),
      scratch_types=[
          pltpu.VMEM((2, 2, CHUNK), jnp.int32),
          pltpu.VMEM((CHUNK, 128), jnp.float32),
          pltpu.VMEM_SHARED((NR, 128), jnp.float32),
      ] + [pltpu.SemaphoreType.DMA] * 2,
      mesh=mesh,
  )
  def deg(dstg, ones, zrows, out, dstb, onesb, dacc, ssem0, ssem1):
    c = lax.axis_index("c")
    s = lax.axis_index("s")
    w = c * NTILE + s
    rbase = pl.multiple_of(s * TROWS, TROWS)
    pltpu.sync_copy(dstg.at[w, 0], dstb.at[0])
    pltpu.sync_copy(ones, onesb)
    pltpu.sync_copy(zrows, dacc.at[pl.ds(rbase, TROWS)])
    plsc.subcore_barrier()

    def body(t, carry):
      p = lax.rem(t, 2)
      s0 = pltpu.async_copy(onesb, dacc.at[dstb.at[p, 0]], ssem0, add=True)
      s1 = pltpu.async_copy(onesb, dacc.at[dstb.at[p, 1]], ssem1, add=True)

      @pl.when(t + 1 < T32)
      def _():
        pltpu.sync_copy(dstg.at[w, t + 1], dstb.at[1 - p])

      s0.wait()
      s1.wait()
      return carry

    lax.fori_loop(0, T32, body, 0)
    plsc.subcore_barrier()
    pltpu.sync_copy(dacc.at[pl.ds(rbase, TROWS)], out.at[c, pl.ds(rbase, TROWS)])

  return deg


def _tc_first(x, w1, degp):
  nprt = degp.shape[1]

  def body(x_ref, w_ref, d_ref, s_ref, dinv_ref):
    d = jnp.sum(d_ref[...], axis=1, keepdims=True) + 1.0
    dinv = lax.rsqrt(d)
    h = jnp.dot(x_ref[...], w_ref[...], preferred_element_type=jnp.float32)
    sv = dinv * h
    s_ref[0] = sv[:, :128]
    s_ref[1] = sv[:, 128:]
    dinv_ref[...] = dinv

  return pl.pallas_call(
      body,
      grid=(GRID,),
      in_specs=[
          pl.BlockSpec((BN, 128), lambda i: (i, 0)),
          pl.BlockSpec((128, 256), lambda i: (0, 0)),
          pl.BlockSpec((BN, nprt), lambda i: (i, 0)),
      ],
      out_specs=[
          pl.BlockSpec((NSC, BN, 128), lambda i: (0, i, 0)),
          pl.BlockSpec((BN, 1), lambda i: (i, 0)),
      ],
      out_shape=[
          jax.ShapeDtypeStruct((NSC, NR, 128), jnp.float32),
          jax.ShapeDtypeStruct((NR, 1), jnp.float32),
      ],
  )(x, w1, degp)


def _tc_mid(agg, sprev, dinv, b, w, hout, split_out):
  hh_in = agg.shape[2]
  hin2 = 2 * hh_in
  hh_out = hout // 2

  def body(a_ref, s_ref, d_ref, b_ref, w_ref, o_ref):
    af = jnp.concatenate([a_ref[0], a_ref[1]], axis=1)
    sf = jnp.concatenate([s_ref[0], s_ref[1]], axis=1)
    dv = d_ref[...]
    h = jnp.maximum(dv * (af + sf) + b_ref[...], 0.0)
    sv = dv * jnp.dot(h, w_ref[...], preferred_element_type=jnp.float32)
    if split_out:
      o_ref[0] = sv[:, :hh_out]
      o_ref[1] = sv[:, hh_out:]
    else:
      o_ref[...] = sv

  if split_out:
    out_spec = pl.BlockSpec((NSC, BN, hh_out), lambda i: (0, i, 0))
    out_shape = jax.ShapeDtypeStruct((NSC, NR, hh_out), jnp.float32)
  else:
    out_spec = pl.BlockSpec((BN, hout), lambda i: (i, 0))
    out_shape = jax.ShapeDtypeStruct((NR, hout), jnp.float32)

  return pl.pallas_call(
      body,
      grid=(GRID,),
      in_specs=[
          pl.BlockSpec((NSC, BN, hh_in), lambda i: (0, i, 0)),
          pl.BlockSpec((NSC, BN, hh_in), lambda i: (0, i, 0)),
          pl.BlockSpec((BN, 1), lambda i: (i, 0)),
          pl.BlockSpec((1, hin2), lambda i: (0, 0)),
          pl.BlockSpec((hin2, hout), lambda i: (0, 0)),
      ],
      out_specs=out_spec,
      out_shape=out_shape,
  )(agg, sprev, dinv, b, w)


def _tc_final(agg, sprev, dinv, b):
  def body(a_ref, s_ref, d_ref, b_ref, o_ref):
    z = d_ref[...] * (a_ref[0] + a_ref[1] + s_ref[...]) + b_ref[...]
    o_ref[...] = jax.nn.sigmoid(jnp.maximum(z, 0.0))

  return pl.pallas_call(
      body,
      grid=(GRID,),
      in_specs=[
          pl.BlockSpec((NSC, BN, 128), lambda i: (0, i, 0)),
          pl.BlockSpec((BN, 128), lambda i: (i, 0)),
          pl.BlockSpec((BN, 1), lambda i: (i, 0)),
          pl.BlockSpec((1, 128), lambda i: (0, 0)),
      ],
      out_specs=pl.BlockSpec((BN, 128), lambda i: (i, 0)),
      out_shape=jax.ShapeDtypeStruct((NR, 128), jnp.float32),
  )(agg, sprev, dinv, b)


@jax.jit
def kernel(x, edge_index, y, W1, b1, W2, b2, W3, b3):
  del y
  src = edge_index[0].astype(jnp.int32)
  dst = edge_index[1].astype(jnp.int32)
  srcp = jnp.zeros((EPAD,), jnp.int32).at[:E].set(src)
  dstp = jnp.full((EPAD,), N, jnp.int32).at[:E].set(dst)
  srcab4 = jnp.concatenate([srcp, srcp + NR]).reshape(NSC, NTILE, T16, 2, CHUNK)
  dst3 = dstp.reshape(NTILE, T16, 2, CHUNK)
  src32 = srcp.reshape(NSC * NTILE, T32, 2, CHUNK)
  dst32 = dstp.reshape(NSC * NTILE, T32, 2, CHUNK)
  ones128 = jnp.ones((CHUNK, 128), jnp.float32)
  z128 = jnp.zeros((TROWS, 128), jnp.float32)
  xp = jnp.zeros((NR, 128), x.dtype).at[:N].set(x)

  hist = _make_deg()(dst32, ones128, z128)
  degp = hist[:, :, 0].T

  agg_split = _make_agg(True)
  s1, dinv = _tc_first(xp, W1, degp)
  agg1 = agg_split(s1.reshape(NSC * NR, 128), srcab4, dst3, z128)
  s2 = _tc_mid(agg1, s1, dinv, b1.reshape(1, 256), W2, 256, True)
  agg2 = agg_split(s2.reshape(NSC * NR, 128), srcab4, dst3, z128)
  s3 = _tc_mid(agg2, s2, dinv, b2.reshape(1, 256), W3, 128, False)
  agg3 = _make_agg(False)(s3, src32, dst32, z128)
  out = _tc_final(agg3, s3, dinv, b3.reshape(1, 128))
  return out[:N]


# speedup vs baseline: 1.0850x; 1.0850x over previous
import functools

import jax
import jax.numpy as jnp
from jax import lax
from jax.experimental import pallas as pl
from jax.experimental.pallas import tpu as pltpu
from jax.experimental.pallas import tpu_sc as plsc

N = 10000
NR = 10240
E = 320000
NSC = 2
NTILE = 16
NW = NSC * NTILE
CHUNK = 128

G = 8
NCH16 = 160
NCH32 = 80
T16 = NCH16 // G
T32 = NCH32 // G
EPT16 = NCH16 * CHUNK
EPAD = EPT16 * NTILE

TROWS = NR // NTILE

BN = 2048
GRID = NR // BN


@functools.lru_cache(maxsize=None)
def _make_agg(split):
  nit = T16 if split else T32
  mesh = plsc.VectorSubcoreMesh(core_axis_name="c", subcore_axis_name="s")

  @functools.partial(
      pl.kernel,
      out_type=jax.ShapeDtypeStruct((NSC, NR, 128), jnp.float32),
      scratch_types=[
          pltpu.VMEM((2, G, CHUNK), jnp.int32),
          pltpu.VMEM((2, G, CHUNK), jnp.int32),
          pltpu.VMEM((2, CHUNK, 128), jnp.float32),
          pltpu.VMEM_SHARED((NR, 128), jnp.float32),
      ] + [pltpu.SemaphoreType.DMA] * 6,
      mesh=mesh,
  )
  def agg(table, srcg, dstg, zrows, out, srcb, dstb, rowb, acc,
          isem0, isem1, gsem0, gsem1, ssem0, ssem1):
    gsem = (gsem0, gsem1)
    ssem = (ssem0, ssem1)
    c = lax.axis_index("c")
    s = lax.axis_index("s")
    rbase = pl.multiple_of(s * TROWS, TROWS)

    if split:
      src_it = lambda t: srcg.at[c, s, t]
      dst_it = lambda t: dstg.at[s, t]
    else:
      w = c * NTILE + s
      src_it = lambda t: srcg.at[w, t]
      dst_it = lambda t: dstg.at[w, t]

    pltpu.sync_copy(src_it(0), srcb.at[0])
    pltpu.sync_copy(dst_it(0), dstb.at[0])
    pltpu.sync_copy(zrows, acc.at[pl.ds(rbase, TROWS)])
    plsc.subcore_barrier()

    def body(t, carry):
      p = lax.rem(t, 2)
      @pl.when(t + 1 < nit)
      def _():
        pltpu.async_copy(src_it(t + 1), srcb.at[1 - p], isem0)
        pltpu.async_copy(dst_it(t + 1), dstb.at[1 - p], isem1)

      g = [None, None]
      g[0] = pltpu.async_copy(table.at[srcb.at[p, 0]], rowb.at[0], gsem[0])
      g[1] = pltpu.async_copy(table.at[srcb.at[p, 1]], rowb.at[1], gsem[1])
      sc = [None, None]
      for j in range(G):
        b = j % 2
        g[b].wait()
        sc[b] = pltpu.async_copy(rowb.at[b], acc.at[dstb.at[p, j]], ssem[b],
                                 add=True)
        if j + 2 < G:
          sc[b].wait()
          g[b] = pltpu.async_copy(table.at[srcb.at[p, j + 2]], rowb.at[b],
                                  gsem[b])
      sc[0].wait()
      sc[1].wait()

      @pl.when(t + 1 < nit)
      def _():
        pltpu.make_async_copy(src_it(t + 1), srcb.at[1 - p], isem0).wait()
        pltpu.make_async_copy(dst_it(t + 1), dstb.at[1 - p], isem1).wait()
      return carry

    lax.fori_loop(0, nit, body, 0)
    plsc.subcore_barrier()
    pltpu.sync_copy(acc.at[pl.ds(rbase, TROWS)], out.at[c, pl.ds(rbase, TROWS)])

  return agg


@functools.lru_cache(maxsize=None)
def _make_deg():
  mesh = plsc.VectorSubcoreMesh(core_axis_name="c", subcore_axis_name="s")

  @functools.partial(
      pl.kernel,
      out_type=jax.ShapeDtypeStruct((NSC, NR, 128), jnp.float32),
      scratch_types=[
          pltpu.VMEM((2, G, CHUNK), jnp.int32),
          pltpu.VMEM((CHUNK, 128), jnp.float32),
          pltpu.VMEM_SHARED((NR, 128), jnp.float32),
      ] + [pltpu.SemaphoreType.DMA] * 3,
      mesh=mesh,
  )
  def deg(dstg, ones, zrows, out, dstb, onesb, dacc, isem, ssem0, ssem1):
    ssem = (ssem0, ssem1)
    c = lax.axis_index("c")
    s = lax.axis_index("s")
    w = c * NTILE + s
    rbase = pl.multiple_of(s * TROWS, TROWS)
    pltpu.sync_copy(dstg.at[w, 0], dstb.at[0])
    pltpu.sync_copy(ones, onesb)
    pltpu.sync_copy(zrows, dacc.at[pl.ds(rbase, TROWS)])
    plsc.subcore_barrier()

    def body(t, carry):
      p = lax.rem(t, 2)

      @pl.when(t + 1 < T32)
      def _():
        pltpu.async_copy(dstg.at[w, t + 1], dstb.at[1 - p], isem)

      sc = [None, None]
      for j in range(G):
        b = j % 2
        if sc[b] is not None:
          sc[b].wait()
        sc[b] = pltpu.async_copy(onesb, dacc.at[dstb.at[p, j]], ssem[b],
                                 add=True)
      sc[0].wait()
      sc[1].wait()

      @pl.when(t + 1 < T32)
      def _():
        pltpu.make_async_copy(dstg.at[w, t + 1], dstb.at[1 - p], isem).wait()
      return carry

    lax.fori_loop(0, T32, body, 0)
    plsc.subcore_barrier()
    pltpu.sync_copy(dacc.at[pl.ds(rbase, TROWS)], out.at[c, pl.ds(rbase, TROWS)])

  return deg


def _tc_first(x, w1, degp):
  nprt = degp.shape[1]

  def body(x_ref, w_ref, d_ref, s_ref, dinv_ref):
    d = jnp.sum(d_ref[...], axis=1, keepdims=True) + 1.0
    dinv = lax.rsqrt(d)
    h = jnp.dot(x_ref[...], w_ref[...], preferred_element_type=jnp.float32)
    sv = dinv * h
    s_ref[0] = sv[:, :128]
    s_ref[1] = sv[:, 128:]
    dinv_ref[...] = dinv

  return pl.pallas_call(
      body,
      grid=(GRID,),
      in_specs=[
          pl.BlockSpec((BN, 128), lambda i: (i, 0)),
          pl.BlockSpec((128, 256), lambda i: (0, 0)),
          pl.BlockSpec((BN, nprt), lambda i: (i, 0)),
      ],
      out_specs=[
          pl.BlockSpec((NSC, BN, 128), lambda i: (0, i, 0)),
          pl.BlockSpec((BN, 1), lambda i: (i, 0)),
      ],
      out_shape=[
          jax.ShapeDtypeStruct((NSC, NR, 128), jnp.float32),
          jax.ShapeDtypeStruct((NR, 1), jnp.float32),
      ],
  )(x, w1, degp)


def _tc_mid(agg, sprev, dinv, b, w, hout, split_out):
  hh_in = agg.shape[2]
  hin2 = 2 * hh_in
  hh_out = hout // 2

  def body(a_ref, s_ref, d_ref, b_ref, w_ref, o_ref):
    af = jnp.concatenate([a_ref[0], a_ref[1]], axis=1)
    sf = jnp.concatenate([s_ref[0], s_ref[1]], axis=1)
    dv = d_ref[...]
    h = jnp.maximum(dv * (af + sf) + b_ref[...], 0.0)
    sv = dv * jnp.dot(h, w_ref[...], preferred_element_type=jnp.float32)
    if split_out:
      o_ref[0] = sv[:, :hh_out]
      o_ref[1] = sv[:, hh_out:]
    else:
      o_ref[...] = sv

  if split_out:
    out_spec = pl.BlockSpec((NSC, BN, hh_out), lambda i: (0, i, 0))
    out_shape = jax.ShapeDtypeStruct((NSC, NR, hh_out), jnp.float32)
  else:
    out_spec = pl.BlockSpec((BN, hout), lambda i: (i, 0))
    out_shape = jax.ShapeDtypeStruct((NR, hout), jnp.float32)

  return pl.pallas_call(
      body,
      grid=(GRID,),
      in_specs=[
          pl.BlockSpec((NSC, BN, hh_in), lambda i: (0, i, 0)),
          pl.BlockSpec((NSC, BN, hh_in), lambda i: (0, i, 0)),
          pl.BlockSpec((BN, 1), lambda i: (i, 0)),
          pl.BlockSpec((1, hin2), lambda i: (0, 0)),
          pl.BlockSpec((hin2, hout), lambda i: (0, 0)),
      ],
      out_specs=out_spec,
      out_shape=out_shape,
  )(agg, sprev, dinv, b, w)


def _tc_final(agg, sprev, dinv, b):
  def body(a_ref, s_ref, d_ref, b_ref, o_ref):
    z = d_ref[...] * (a_ref[0] + a_ref[1] + s_ref[...]) + b_ref[...]
    o_ref[...] = jax.nn.sigmoid(jnp.maximum(z, 0.0))

  return pl.pallas_call(
      body,
      grid=(GRID,),
      in_specs=[
          pl.BlockSpec((NSC, BN, 128), lambda i: (0, i, 0)),
          pl.BlockSpec((BN, 128), lambda i: (i, 0)),
          pl.BlockSpec((BN, 1), lambda i: (i, 0)),
          pl.BlockSpec((1, 128), lambda i: (0, 0)),
      ],
      out_specs=pl.BlockSpec((BN, 128), lambda i: (i, 0)),
      out_shape=jax.ShapeDtypeStruct((NR, 128), jnp.float32),
  )(agg, sprev, dinv, b)


@jax.jit
def kernel(x, edge_index, y, W1, b1, W2, b2, W3, b3):
  del y
  src = edge_index[0].astype(jnp.int32)
  dst = edge_index[1].astype(jnp.int32)
  srcp = jnp.zeros((EPAD,), jnp.int32).at[:E].set(src)
  dstp = jnp.full((EPAD,), N, jnp.int32).at[:E].set(dst)
  srcab4 = jnp.concatenate([srcp, srcp + NR]).reshape(NSC, NTILE, T16, G, CHUNK)
  dst3 = dstp.reshape(NTILE, T16, G, CHUNK)
  src32 = srcp.reshape(NW, T32, G, CHUNK)
  dst32 = dstp.reshape(NW, T32, G, CHUNK)
  ones128 = jnp.ones((CHUNK, 128), jnp.float32)
  z128 = jnp.zeros((TROWS, 128), jnp.float32)
  xp = jnp.zeros((NR, 128), x.dtype).at[:N].set(x)

  hist = _make_deg()(dst32, ones128, z128)
  degp = hist[:, :, 0].T

  agg_split = _make_agg(True)
  s1, dinv = _tc_first(xp, W1, degp)
  agg1 = agg_split(s1.reshape(NSC * NR, 128), srcab4, dst3, z128)
  s2 = _tc_mid(agg1, s1, dinv, b1.reshape(1, 256), W2, 256, True)
  agg2 = agg_split(s2.reshape(NSC * NR, 128), srcab4, dst3, z128)
  s3 = _tc_mid(agg2, s2, dinv, b2.reshape(1, 256), W3, 128, False)
  agg3 = _make_agg(False)(s3, src32, dst32, z128)
  out = _tc_final(agg3, s3, dinv, b3.reshape(1, 128))
  return out[:N]


# speedup vs baseline: 1.2273x; 1.1312x over previous
import functools

import jax
import jax.numpy as jnp
from jax import lax
from jax.experimental import pallas as pl
from jax.experimental.pallas import tpu as pltpu
from jax.experimental.pallas import tpu_sc as plsc

N = 10000
NR = 10240
E = 320000
NSC = 2
NTILE = 16
NW = NSC * NTILE
CHUNK = 128

NCH16 = 157
EPT16 = NCH16 * CHUNK
EPAD16 = EPT16 * NTILE

NCH32 = 79
EPT32 = NCH32 * CHUNK
EPAD32 = EPT32 * NW

G = 8
T32 = 10
EPTD = T32 * G * CHUNK
EPADD = EPTD * NW

TROWS = NR // NTILE

BN = 2048
GRID = NR // BN


@functools.lru_cache(maxsize=None)
def _make_agg(hh):
  mesh = plsc.VectorSubcoreMesh(core_axis_name="c", subcore_axis_name="s")

  @functools.partial(
      pl.kernel,
      out_type=jax.ShapeDtypeStruct((NSC, NR, hh), jnp.float32),
      scratch_types=[
          pltpu.VMEM((CHUNK,), jnp.int32),
          pltpu.VMEM((CHUNK,), jnp.int32),
          pltpu.VMEM((CHUNK, hh), jnp.float32),
          pltpu.VMEM_SHARED((NR, hh), jnp.float32),
          pltpu.SemaphoreType.DMA,
      ],
      mesh=mesh,
  )
  def agg(table, srcab, dstp, zrows, out, idxb, dstb, rowb, acc, sem):
    c = lax.axis_index("c")
    s = lax.axis_index("s")
    rbase = pl.multiple_of(s * TROWS, TROWS)
    pltpu.sync_copy(zrows, acc.at[pl.ds(rbase, TROWS)])
    plsc.subcore_barrier()
    ebase = s * EPT16

    def body(k, carry):
      off = pl.multiple_of(ebase + k * CHUNK, CHUNK)
      soff = pl.multiple_of(c * EPAD16 + off, CHUNK)
      pltpu.sync_copy(srcab.at[pl.ds(soff, CHUNK)], idxb)
      pltpu.sync_copy(dstp.at[pl.ds(off, CHUNK)], dstb)
      pltpu.async_copy(table.at[idxb], rowb, sem).wait()
      pltpu.sync_copy(rowb, acc.at[dstb], add=True)
      return carry

    lax.fori_loop(0, NCH16, body, 0)
    plsc.subcore_barrier()
    pltpu.sync_copy(acc.at[pl.ds(rbase, TROWS)], out.at[c, pl.ds(rbase, TROWS)])

  return agg


@functools.lru_cache(maxsize=None)
def _make_agg_part():
  mesh = plsc.VectorSubcoreMesh(core_axis_name="c", subcore_axis_name="s")

  @functools.partial(
      pl.kernel,
      out_type=jax.ShapeDtypeStruct((NSC, NR, 128), jnp.float32),
      scratch_types=[
          pltpu.VMEM((CHUNK,), jnp.int32),
          pltpu.VMEM((CHUNK,), jnp.int32),
          pltpu.VMEM((CHUNK, 128), jnp.float32),
          pltpu.VMEM_SHARED((NR, 128), jnp.float32),
          pltpu.SemaphoreType.DMA,
      ],
      mesh=mesh,
  )
  def agg(table, srcp, dstp, zrows, out, idxb, dstb, rowb, acc, sem):
    c = lax.axis_index("c")
    s = lax.axis_index("s")
    rbase = pl.multiple_of(s * TROWS, TROWS)
    pltpu.sync_copy(zrows, acc.at[pl.ds(rbase, TROWS)])
    plsc.subcore_barrier()
    ebase = (c * NTILE + s) * EPT32

    def body(k, carry):
      off = pl.multiple_of(ebase + k * CHUNK, CHUNK)
      pltpu.sync_copy(srcp.at[pl.ds(off, CHUNK)], idxb)
      pltpu.sync_copy(dstp.at[pl.ds(off, CHUNK)], dstb)
      pltpu.async_copy(table.at[idxb], rowb, sem).wait()
      pltpu.sync_copy(rowb, acc.at[dstb], add=True)
      return carry

    lax.fori_loop(0, NCH32, body, 0)
    plsc.subcore_barrier()
    pltpu.sync_copy(acc.at[pl.ds(rbase, TROWS)], out.at[c, pl.ds(rbase, TROWS)])

  return agg


@functools.lru_cache(maxsize=None)
def _make_deg():
  mesh = plsc.VectorSubcoreMesh(core_axis_name="c", subcore_axis_name="s")

  @functools.partial(
      pl.kernel,
      out_type=jax.ShapeDtypeStruct((NSC, NR, 128), jnp.float32),
      scratch_types=[
          pltpu.VMEM((2, G, CHUNK), jnp.int32),
          pltpu.VMEM((CHUNK, 128), jnp.float32),
          pltpu.VMEM_SHARED((NR, 128), jnp.float32),
      ] + [pltpu.SemaphoreType.DMA] * 3,
      mesh=mesh,
  )
  def deg(dstg, ones, zrows, out, dstb, onesb, dacc, isem, ssem0, ssem1):
    ssem = (ssem0, ssem1)
    c = lax.axis_index("c")
    s = lax.axis_index("s")
    w = c * NTILE + s
    rbase = pl.multiple_of(s * TROWS, TROWS)
    pltpu.sync_copy(dstg.at[w, 0], dstb.at[0])
    pltpu.sync_copy(ones, onesb)
    pltpu.sync_copy(zrows, dacc.at[pl.ds(rbase, TROWS)])
    plsc.subcore_barrier()

    def body(t, carry):
      p = lax.rem(t, 2)

      @pl.when(t + 1 < T32)
      def _():
        pltpu.async_copy(dstg.at[w, t + 1], dstb.at[1 - p], isem)

      sc = [None, None]
      for j in range(G):
        b = j % 2
        if sc[b] is not None:
          sc[b].wait()
        sc[b] = pltpu.async_copy(onesb, dacc.at[dstb.at[p, j]], ssem[b],
                                 add=True)
      sc[0].wait()
      sc[1].wait()

      @pl.when(t + 1 < T32)
      def _():
        pltpu.make_async_copy(dstg.at[w, t + 1], dstb.at[1 - p], isem).wait()
      return carry

    lax.fori_loop(0, T32, body, 0)
    plsc.subcore_barrier()
    pltpu.sync_copy(dacc.at[pl.ds(rbase, TROWS)], out.at[c, pl.ds(rbase, TROWS)])

  return deg


def _tc_first(x, w1, degp):
  nprt = degp.shape[1]

  def body(x_ref, w_ref, d_ref, s_ref, dinv_ref):
    d = jnp.sum(d_ref[...], axis=1, keepdims=True) + 1.0
    dinv = lax.rsqrt(d)
    h = jnp.dot(x_ref[...], w_ref[...], preferred_element_type=jnp.float32)
    sv = dinv * h
    s_ref[0] = sv[:, :128]
    s_ref[1] = sv[:, 128:]
    dinv_ref[...] = dinv

  return pl.pallas_call(
      body,
      grid=(GRID,),
      in_specs=[
          pl.BlockSpec((BN, 128), lambda i: (i, 0)),
          pl.BlockSpec((128, 256), lambda i: (0, 0)),
          pl.BlockSpec((BN, nprt), lambda i: (i, 0)),
      ],
      out_specs=[
          pl.BlockSpec((NSC, BN, 128), lambda i: (0, i, 0)),
          pl.BlockSpec((BN, 1), lambda i: (i, 0)),
      ],
      out_shape=[
          jax.ShapeDtypeStruct((NSC, NR, 128), jnp.float32),
          jax.ShapeDtypeStruct((NR, 1), jnp.float32),
      ],
  )(x, w1, degp)


def _tc_mid(agg, sprev, dinv, b, w, hout, split_out):
  hh_in = agg.shape[2]
  hin2 = 2 * hh_in
  hh_out = hout // 2

  def body(a_ref, s_ref, d_ref, b_ref, w_ref, o_ref):
    af = jnp.concatenate([a_ref[0], a_ref[1]], axis=1)
    sf = jnp.concatenate([s_ref[0], s_ref[1]], axis=1)
    dv = d_ref[...]
    h = jnp.maximum(dv * (af + sf) + b_ref[...], 0.0)
    sv = dv * jnp.dot(h, w_ref[...], preferred_element_type=jnp.float32)
    if split_out:
      o_ref[0] = sv[:, :hh_out]
      o_ref[1] = sv[:, hh_out:]
    else:
      o_ref[...] = sv

  if split_out:
    out_spec = pl.BlockSpec((NSC, BN, hh_out), lambda i: (0, i, 0))
    out_shape = jax.ShapeDtypeStruct((NSC, NR, hh_out), jnp.float32)
  else:
    out_spec = pl.BlockSpec((BN, hout), lambda i: (i, 0))
    out_shape = jax.ShapeDtypeStruct((NR, hout), jnp.float32)

  return pl.pallas_call(
      body,
      grid=(GRID,),
      in_specs=[
          pl.BlockSpec((NSC, BN, hh_in), lambda i: (0, i, 0)),
          pl.BlockSpec((NSC, BN, hh_in), lambda i: (0, i, 0)),
          pl.BlockSpec((BN, 1), lambda i: (i, 0)),
          pl.BlockSpec((1, hin2), lambda i: (0, 0)),
          pl.BlockSpec((hin2, hout), lambda i: (0, 0)),
      ],
      out_specs=out_spec,
      out_shape=out_shape,
  )(agg, sprev, dinv, b, w)


def _tc_final(agg, sprev, dinv, b):
  def body(a_ref, s_ref, d_ref, b_ref, o_ref):
    z = d_ref[...] * (a_ref[0] + a_ref[1] + s_ref[...]) + b_ref[...]
    o_ref[...] = jax.nn.sigmoid(jnp.maximum(z, 0.0))

  return pl.pallas_call(
      body,
      grid=(GRID,),
      in_specs=[
          pl.BlockSpec((NSC, BN, 128), lambda i: (0, i, 0)),
          pl.BlockSpec((BN, 128), lambda i: (i, 0)),
          pl.BlockSpec((BN, 1), lambda i: (i, 0)),
          pl.BlockSpec((1, 128), lambda i: (0, 0)),
      ],
      out_specs=pl.BlockSpec((BN, 128), lambda i: (i, 0)),
      out_shape=jax.ShapeDtypeStruct((NR, 128), jnp.float32),
  )(agg, sprev, dinv, b)


@jax.jit
def kernel(x, edge_index, y, W1, b1, W2, b2, W3, b3):
  del y
  src = edge_index[0].astype(jnp.int32)
  dst = edge_index[1].astype(jnp.int32)
  srcp = jnp.zeros((EPAD16,), jnp.int32).at[:E].set(src)
  srcab = jnp.concatenate([srcp, srcp + NR])
  dstp16 = jnp.full((EPAD16,), N, jnp.int32).at[:E].set(dst)
  srcp32 = jnp.zeros((EPAD32,), jnp.int32).at[:E].set(src)
  dstp32 = jnp.full((EPAD32,), N, jnp.int32).at[:E].set(dst)
  dstd = jnp.full((EPADD,), N, jnp.int32).at[:E].set(dst)
  dstg = dstd.reshape(NW, T32, G, CHUNK)
  ones128 = jnp.ones((CHUNK, 128), jnp.float32)
  z128 = jnp.zeros((TROWS, 128), jnp.float32)
  xp = jnp.zeros((NR, 128), x.dtype).at[:N].set(x)

  hist = _make_deg()(dstg, ones128, z128)
  degp = hist[:, :, 0].T

  agg128 = _make_agg(128)
  s1, dinv = _tc_first(xp, W1, degp)
  agg1 = agg128(s1.reshape(NSC * NR, 128), srcab, dstp16, z128)
  s2 = _tc_mid(agg1, s1, dinv, b1.reshape(1, 256), W2, 256, True)
  agg2 = agg128(s2.reshape(NSC * NR, 128), srcab, dstp16, z128)
  s3 = _tc_mid(agg2, s2, dinv, b2.reshape(1, 256), W3, 128, False)
  agg3 = _make_agg_part()(s3, srcp32, dstp32, z128)
  out = _tc_final(agg3, s3, dinv, b3.reshape(1, 128))
  return out[:N]


# speedup vs baseline: 1.5749x; 1.2832x over previous
import functools

import jax
import jax.numpy as jnp
from jax import lax
from jax.experimental import pallas as pl
from jax.experimental.pallas import tpu as pltpu
from jax.experimental.pallas import tpu_sc as plsc

N = 10000
NR = 10240
E = 320000
NSC = 2
NTILE = 16
NW = NSC * NTILE
CHUNK = 128

NCH16 = 157
EPT16 = NCH16 * CHUNK
EPAD16 = EPT16 * NTILE

NCH32 = 79
EPT32 = NCH32 * CHUNK
EPAD32 = EPT32 * NW

G = 8
T32 = 10
EPTD = T32 * G * CHUNK
EPADD = EPTD * NW

TROWS = NR // NTILE

BN = 2048
GRID = NR // BN


@functools.lru_cache(maxsize=None)
def _make_agg(hh):
  mesh = plsc.VectorSubcoreMesh(core_axis_name="c", subcore_axis_name="s")

  @functools.partial(
      pl.kernel,
      out_type=jax.ShapeDtypeStruct((NSC, NR, hh), jnp.float32),
      scratch_types=[
          pltpu.VMEM((CHUNK,), jnp.int32),
          pltpu.VMEM((CHUNK,), jnp.int32),
          pltpu.VMEM((CHUNK,), jnp.int32),
          pltpu.VMEM((CHUNK,), jnp.int32),
          pltpu.VMEM((CHUNK, hh), jnp.float32),
          pltpu.VMEM((CHUNK, hh), jnp.float32),
          pltpu.VMEM_SHARED((NR, hh), jnp.float32),
          pltpu.SemaphoreType.DMA,
          pltpu.SemaphoreType.DMA,
      ],
      mesh=mesh,
  )
  def agg(table, srcab, dstp, zrows, out, idxb0, idxb1, dstb0, dstb1,
          rowb0, rowb1, acc, gsem0, gsem1):
    idxbs = (idxb0, idxb1)
    dstbs = (dstb0, dstb1)
    rowbs = (rowb0, rowb1)
    gsems = (gsem0, gsem1)
    c = lax.axis_index("c")
    s = lax.axis_index("s")
    rbase = pl.multiple_of(s * TROWS, TROWS)
    pltpu.sync_copy(zrows, acc.at[pl.ds(rbase, TROWS)])
    ebase = s * EPT16
    sbase = c * EPAD16 + ebase

    def load_and_gather(k, slot):
      off = pl.multiple_of(ebase + k * CHUNK, CHUNK)
      soff = pl.multiple_of(sbase + k * CHUNK, CHUNK)
      pltpu.sync_copy(srcab.at[pl.ds(soff, CHUNK)], idxbs[slot])
      pltpu.sync_copy(dstp.at[pl.ds(off, CHUNK)], dstbs[slot])
      pltpu.async_copy(table.at[idxbs[slot]], rowbs[slot], gsems[slot])

    plsc.subcore_barrier()
    load_and_gather(0, 0)

    def body(k, carry):
      p = lax.rem(k, 2)
      for pp in (0, 1):
        @pl.when(p == pp)
        def _(pp=pp):
          me, ot = pp, 1 - pp
          @pl.when(k + 1 < NCH16)
          def _():
            load_and_gather(k + 1, ot)
          pltpu.make_async_copy(table.at[idxbs[me]], rowbs[me],
                                gsems[me]).wait()
          pltpu.sync_copy(rowbs[me], acc.at[dstbs[me]], add=True)
      return carry

    lax.fori_loop(0, NCH16, body, 0)
    plsc.subcore_barrier()
    pltpu.sync_copy(acc.at[pl.ds(rbase, TROWS)], out.at[c, pl.ds(rbase, TROWS)])

  return agg


@functools.lru_cache(maxsize=None)
def _make_agg_part():
  mesh = plsc.VectorSubcoreMesh(core_axis_name="c", subcore_axis_name="s")

  @functools.partial(
      pl.kernel,
      out_type=jax.ShapeDtypeStruct((NSC, NR, 128), jnp.float32),
      scratch_types=[
          pltpu.VMEM((CHUNK,), jnp.int32),
          pltpu.VMEM((CHUNK,), jnp.int32),
          pltpu.VMEM((CHUNK, 128), jnp.float32),
          pltpu.VMEM_SHARED((NR, 128), jnp.float32),
          pltpu.SemaphoreType.DMA,
      ],
      mesh=mesh,
  )
  def agg(table, srcp, dstp, zrows, out, idxb, dstb, rowb, acc, sem):
    c = lax.axis_index("c")
    s = lax.axis_index("s")
    rbase = pl.multiple_of(s * TROWS, TROWS)
    pltpu.sync_copy(zrows, acc.at[pl.ds(rbase, TROWS)])
    plsc.subcore_barrier()
    ebase = (c * NTILE + s) * EPT32

    def body(k, carry):
      off = pl.multiple_of(ebase + k * CHUNK, CHUNK)
      pltpu.sync_copy(srcp.at[pl.ds(off, CHUNK)], idxb)
      pltpu.sync_copy(dstp.at[pl.ds(off, CHUNK)], dstb)
      pltpu.async_copy(table.at[idxb], rowb, sem).wait()
      pltpu.sync_copy(rowb, acc.at[dstb], add=True)
      return carry

    lax.fori_loop(0, NCH32, body, 0)
    plsc.subcore_barrier()
    pltpu.sync_copy(acc.at[pl.ds(rbase, TROWS)], out.at[c, pl.ds(rbase, TROWS)])

  return agg


@functools.lru_cache(maxsize=None)
def _make_deg():
  mesh = plsc.VectorSubcoreMesh(core_axis_name="c", subcore_axis_name="s")

  @functools.partial(
      pl.kernel,
      out_type=jax.ShapeDtypeStruct((NSC, NR, 128), jnp.float32),
      scratch_types=[
          pltpu.VMEM((2, G, CHUNK), jnp.int32),
          pltpu.VMEM((CHUNK, 128), jnp.float32),
          pltpu.VMEM_SHARED((NR, 128), jnp.float32),
      ] + [pltpu.SemaphoreType.DMA] * 3,
      mesh=mesh,
  )
  def deg(dstg, ones, zrows, out, dstb, onesb, dacc, isem, ssem0, ssem1):
    ssem = (ssem0, ssem1)
    c = lax.axis_index("c")
    s = lax.axis_index("s")
    w = c * NTILE + s
    rbase = pl.multiple_of(s * TROWS, TROWS)
    pltpu.sync_copy(dstg.at[w, 0], dstb.at[0])
    pltpu.sync_copy(ones, onesb)
    pltpu.sync_copy(zrows, dacc.at[pl.ds(rbase, TROWS)])
    plsc.subcore_barrier()

    def body(t, carry):
      p = lax.rem(t, 2)

      @pl.when(t + 1 < T32)
      def _():
        pltpu.async_copy(dstg.at[w, t + 1], dstb.at[1 - p], isem)

      sc = [None, None]
      for j in range(G):
        b = j % 2
        if sc[b] is not None:
          sc[b].wait()
        sc[b] = pltpu.async_copy(onesb, dacc.at[dstb.at[p, j]], ssem[b],
                                 add=True)
      sc[0].wait()
      sc[1].wait()

      @pl.when(t + 1 < T32)
      def _():
        pltpu.make_async_copy(dstg.at[w, t + 1], dstb.at[1 - p], isem).wait()
      return carry

    lax.fori_loop(0, T32, body, 0)
    plsc.subcore_barrier()
    pltpu.sync_copy(dacc.at[pl.ds(rbase, TROWS)], out.at[c, pl.ds(rbase, TROWS)])

  return deg


def _tc_first(x, w1, degp):
  nprt = degp.shape[1]

  def body(x_ref, w_ref, d_ref, s_ref, dinv_ref):
    d = jnp.sum(d_ref[...], axis=1, keepdims=True) + 1.0
    dinv = lax.rsqrt(d)
    h = jnp.dot(x_ref[...], w_ref[...], preferred_element_type=jnp.float32)
    sv = dinv * h
    s_ref[0] = sv[:, :128]
    s_ref[1] = sv[:, 128:]
    dinv_ref[...] = dinv

  return pl.pallas_call(
      body,
      grid=(GRID,),
      in_specs=[
          pl.BlockSpec((BN, 128), lambda i: (i, 0)),
          pl.BlockSpec((128, 256), lambda i: (0, 0)),
          pl.BlockSpec((BN, nprt), lambda i: (i, 0)),
      ],
      out_specs=[
          pl.BlockSpec((NSC, BN, 128), lambda i: (0, i, 0)),
          pl.BlockSpec((BN, 1), lambda i: (i, 0)),
      ],
      out_shape=[
          jax.ShapeDtypeStruct((NSC, NR, 128), jnp.float32),
          jax.ShapeDtypeStruct((NR, 1), jnp.float32),
      ],
  )(x, w1, degp)


def _tc_mid(agg, sprev, dinv, b, w, hout, split_out):
  hh_in = agg.shape[2]
  hin2 = 2 * hh_in
  hh_out = hout // 2

  def body(a_ref, s_ref, d_ref, b_ref, w_ref, o_ref):
    af = jnp.concatenate([a_ref[0], a_ref[1]], axis=1)
    sf = jnp.concatenate([s_ref[0], s_ref[1]], axis=1)
    dv = d_ref[...]
    h = jnp.maximum(dv * (af + sf) + b_ref[...], 0.0)
    sv = dv * jnp.dot(h, w_ref[...], preferred_element_type=jnp.float32)
    if split_out:
      o_ref[0] = sv[:, :hh_out]
      o_ref[1] = sv[:, hh_out:]
    else:
      o_ref[...] = sv

  if split_out:
    out_spec = pl.BlockSpec((NSC, BN, hh_out), lambda i: (0, i, 0))
    out_shape = jax.ShapeDtypeStruct((NSC, NR, hh_out), jnp.float32)
  else:
    out_spec = pl.BlockSpec((BN, hout), lambda i: (i, 0))
    out_shape = jax.ShapeDtypeStruct((NR, hout), jnp.float32)

  return pl.pallas_call(
      body,
      grid=(GRID,),
      in_specs=[
          pl.BlockSpec((NSC, BN, hh_in), lambda i: (0, i, 0)),
          pl.BlockSpec((NSC, BN, hh_in), lambda i: (0, i, 0)),
          pl.BlockSpec((BN, 1), lambda i: (i, 0)),
          pl.BlockSpec((1, hin2), lambda i: (0, 0)),
          pl.BlockSpec((hin2, hout), lambda i: (0, 0)),
      ],
      out_specs=out_spec,
      out_shape=out_shape,
  )(agg, sprev, dinv, b, w)


def _tc_final(agg, sprev, dinv, b):
  def body(a_ref, s_ref, d_ref, b_ref, o_ref):
    z = d_ref[...] * (a_ref[0] + a_ref[1] + s_ref[...]) + b_ref[...]
    o_ref[...] = jax.nn.sigmoid(jnp.maximum(z, 0.0))

  return pl.pallas_call(
      body,
      grid=(GRID,),
      in_specs=[
          pl.BlockSpec((NSC, BN, 128), lambda i: (0, i, 0)),
          pl.BlockSpec((BN, 128), lambda i: (i, 0)),
          pl.BlockSpec((BN, 1), lambda i: (i, 0)),
          pl.BlockSpec((1, 128), lambda i: (0, 0)),
      ],
      out_specs=pl.BlockSpec((BN, 128), lambda i: (i, 0)),
      out_shape=jax.ShapeDtypeStruct((NR, 128), jnp.float32),
  )(agg, sprev, dinv, b)


@jax.jit
def kernel(x, edge_index, y, W1, b1, W2, b2, W3, b3):
  del y
  src = edge_index[0].astype(jnp.int32)
  dst = edge_index[1].astype(jnp.int32)
  srcp = jnp.zeros((EPAD16,), jnp.int32).at[:E].set(src)
  srcab = jnp.concatenate([srcp, srcp + NR])
  dstp16 = jnp.full((EPAD16,), N, jnp.int32).at[:E].set(dst)
  srcp32 = jnp.zeros((EPAD32,), jnp.int32).at[:E].set(src)
  dstp32 = jnp.full((EPAD32,), N, jnp.int32).at[:E].set(dst)
  dstd = jnp.full((EPADD,), N, jnp.int32).at[:E].set(dst)
  dstg = dstd.reshape(NW, T32, G, CHUNK)
  ones128 = jnp.ones((CHUNK, 128), jnp.float32)
  z128 = jnp.zeros((TROWS, 128), jnp.float32)
  xp = jnp.zeros((NR, 128), x.dtype).at[:N].set(x)

  hist = _make_deg()(dstg, ones128, z128)
  degp = hist[:, :, 0].T

  agg128 = _make_agg(128)
  s1, dinv = _tc_first(xp, W1, degp)
  agg1 = agg128(s1.reshape(NSC * NR, 128), srcab, dstp16, z128)
  s2 = _tc_mid(agg1, s1, dinv, b1.reshape(1, 256), W2, 256, True)
  agg2 = agg128(s2.reshape(NSC * NR, 128), srcab, dstp16, z128)
  s3 = _tc_mid(agg2, s2, dinv, b2.reshape(1, 256), W3, 128, False)
  agg3 = _make_agg_part()(s3, srcp32, dstp32, z128)
  out = _tc_final(agg3, s3, dinv, b3.reshape(1, 128))
  return out[:N]


# speedup vs baseline: 1.7088x; 1.0850x over previous
import functools

import jax
import jax.numpy as jnp
from jax import lax
from jax.experimental import pallas as pl
from jax.experimental.pallas import tpu as pltpu
from jax.experimental.pallas import tpu_sc as plsc

N = 10000
NR = 10240
E = 320000
NSC = 2
NTILE = 16
NW = NSC * NTILE
CHUNK = 128

NCH16 = 157
EPT16 = NCH16 * CHUNK
EPAD16 = EPT16 * NTILE

NCH32 = 79
EPT32 = NCH32 * CHUNK
EPAD32 = EPT32 * NW

G = 8
T32 = 10
EPTD = T32 * G * CHUNK
EPADD = EPTD * NW

TROWS = NR // NTILE

BN = 2048
GRID = NR // BN


@functools.lru_cache(maxsize=None)
def _make_agg(hh):
  mesh = plsc.VectorSubcoreMesh(core_axis_name="c", subcore_axis_name="s")

  @functools.partial(
      pl.kernel,
      out_type=jax.ShapeDtypeStruct((NSC, NR, hh), jnp.float32),
      scratch_types=[
          pltpu.VMEM((CHUNK,), jnp.int32),
          pltpu.VMEM((CHUNK,), jnp.int32),
          pltpu.VMEM((CHUNK,), jnp.int32),
          pltpu.VMEM((CHUNK,), jnp.int32),
          pltpu.VMEM((CHUNK, hh), jnp.float32),
          pltpu.VMEM((CHUNK, hh), jnp.float32),
          pltpu.VMEM_SHARED((NR, hh), jnp.float32),
          pltpu.SemaphoreType.DMA,
          pltpu.SemaphoreType.DMA,
      ],
      mesh=mesh,
  )
  def agg(table, srcab, dstp, zrows, out, idxb0, idxb1, dstb0, dstb1,
          rowb0, rowb1, acc, gsem0, gsem1):
    idxbs = (idxb0, idxb1)
    dstbs = (dstb0, dstb1)
    rowbs = (rowb0, rowb1)
    gsems = (gsem0, gsem1)
    c = lax.axis_index("c")
    s = lax.axis_index("s")
    rbase = pl.multiple_of(s * TROWS, TROWS)
    pltpu.sync_copy(zrows, acc.at[pl.ds(rbase, TROWS)])
    ebase = s * EPT16
    sbase = c * EPAD16 + ebase

    def load_and_gather(k, slot):
      off = pl.multiple_of(ebase + k * CHUNK, CHUNK)
      soff = pl.multiple_of(sbase + k * CHUNK, CHUNK)
      pltpu.sync_copy(srcab.at[pl.ds(soff, CHUNK)], idxbs[slot])
      pltpu.sync_copy(dstp.at[pl.ds(off, CHUNK)], dstbs[slot])
      pltpu.async_copy(table.at[idxbs[slot]], rowbs[slot], gsems[slot])

    plsc.subcore_barrier()
    load_and_gather(0, 0)

    def body(k, carry):
      p = lax.rem(k, 2)
      for pp in (0, 1):
        @pl.when(p == pp)
        def _(pp=pp):
          me, ot = pp, 1 - pp
          @pl.when(k + 1 < NCH16)
          def _():
            load_and_gather(k + 1, ot)
          pltpu.make_async_copy(table.at[idxbs[me]], rowbs[me],
                                gsems[me]).wait()
          pltpu.sync_copy(rowbs[me], acc.at[dstbs[me]], add=True)
      return carry

    lax.fori_loop(0, NCH16, body, 0)
    plsc.subcore_barrier()
    pltpu.sync_copy(acc.at[pl.ds(rbase, TROWS)], out.at[c, pl.ds(rbase, TROWS)])

  return agg


@functools.lru_cache(maxsize=None)
def _make_agg_part():
  mesh = plsc.VectorSubcoreMesh(core_axis_name="c", subcore_axis_name="s")

  @functools.partial(
      pl.kernel,
      out_type=jax.ShapeDtypeStruct((NSC, NR, 128), jnp.float32),
      scratch_types=[
          pltpu.VMEM((CHUNK,), jnp.int32),
          pltpu.VMEM((CHUNK,), jnp.int32),
          pltpu.VMEM((CHUNK,), jnp.int32),
          pltpu.VMEM((CHUNK,), jnp.int32),
          pltpu.VMEM((CHUNK, 128), jnp.float32),
          pltpu.VMEM((CHUNK, 128), jnp.float32),
          pltpu.VMEM_SHARED((NR, 128), jnp.float32),
          pltpu.SemaphoreType.DMA,
          pltpu.SemaphoreType.DMA,
      ],
      mesh=mesh,
  )
  def agg(table, srcp, dstp, zrows, out, idxb0, idxb1, dstb0, dstb1,
          rowb0, rowb1, acc, gsem0, gsem1):
    idxbs = (idxb0, idxb1)
    dstbs = (dstb0, dstb1)
    rowbs = (rowb0, rowb1)
    gsems = (gsem0, gsem1)
    c = lax.axis_index("c")
    s = lax.axis_index("s")
    rbase = pl.multiple_of(s * TROWS, TROWS)
    pltpu.sync_copy(zrows, acc.at[pl.ds(rbase, TROWS)])
    ebase = (c * NTILE + s) * EPT32

    def load_and_gather(k, slot):
      off = pl.multiple_of(ebase + k * CHUNK, CHUNK)
      pltpu.sync_copy(srcp.at[pl.ds(off, CHUNK)], idxbs[slot])
      pltpu.sync_copy(dstp.at[pl.ds(off, CHUNK)], dstbs[slot])
      pltpu.async_copy(table.at[idxbs[slot]], rowbs[slot], gsems[slot])

    plsc.subcore_barrier()
    load_and_gather(0, 0)

    def body(k, carry):
      p = lax.rem(k, 2)
      for pp in (0, 1):
        @pl.when(p == pp)
        def _(pp=pp):
          me, ot = pp, 1 - pp
          @pl.when(k + 1 < NCH32)
          def _():
            load_and_gather(k + 1, ot)
          pltpu.make_async_copy(table.at[idxbs[me]], rowbs[me],
                                gsems[me]).wait()
          pltpu.sync_copy(rowbs[me], acc.at[dstbs[me]], add=True)
      return carry

    lax.fori_loop(0, NCH32, body, 0)
    plsc.subcore_barrier()
    pltpu.sync_copy(acc.at[pl.ds(rbase, TROWS)], out.at[c, pl.ds(rbase, TROWS)])

  return agg


@functools.lru_cache(maxsize=None)
def _make_deg():
  mesh = plsc.VectorSubcoreMesh(core_axis_name="c", subcore_axis_name="s")

  @functools.partial(
      pl.kernel,
      out_type=jax.ShapeDtypeStruct((NSC, NR, 128), jnp.float32),
      scratch_types=[
          pltpu.VMEM((2, G, CHUNK), jnp.int32),
          pltpu.VMEM((CHUNK, 128), jnp.float32),
          pltpu.VMEM_SHARED((NR, 128), jnp.float32),
      ] + [pltpu.SemaphoreType.DMA] * 3,
      mesh=mesh,
  )
  def deg(dstg, ones, zrows, out, dstb, onesb, dacc, isem, ssem0, ssem1):
    ssem = (ssem0, ssem1)
    c = lax.axis_index("c")
    s = lax.axis_index("s")
    w = c * NTILE + s
    rbase = pl.multiple_of(s * TROWS, TROWS)
    pltpu.sync_copy(dstg.at[w, 0], dstb.at[0])
    pltpu.sync_copy(ones, onesb)
    pltpu.sync_copy(zrows, dacc.at[pl.ds(rbase, TROWS)])
    plsc.subcore_barrier()

    def body(t, carry):
      p = lax.rem(t, 2)

      @pl.when(t + 1 < T32)
      def _():
        pltpu.async_copy(dstg.at[w, t + 1], dstb.at[1 - p], isem)

      sc = [None, None]
      for j in range(G):
        b = j % 2
        if sc[b] is not None:
          sc[b].wait()
        sc[b] = pltpu.async_copy(onesb, dacc.at[dstb.at[p, j]], ssem[b],
                                 add=True)
      sc[0].wait()
      sc[1].wait()

      @pl.when(t + 1 < T32)
      def _():
        pltpu.make_async_copy(dstg.at[w, t + 1], dstb.at[1 - p], isem).wait()
      return carry

    lax.fori_loop(0, T32, body, 0)
    plsc.subcore_barrier()
    pltpu.sync_copy(dacc.at[pl.ds(rbase, TROWS)], out.at[c, pl.ds(rbase, TROWS)])

  return deg


def _tc_first(x, w1, degp):
  nprt = degp.shape[1]

  def body(x_ref, w_ref, d_ref, s_ref, dinv_ref):
    d = jnp.sum(d_ref[...], axis=1, keepdims=True) + 1.0
    dinv = lax.rsqrt(d)
    h = jnp.dot(x_ref[...], w_ref[...], preferred_element_type=jnp.float32)
    sv = dinv * h
    s_ref[0] = sv[:, :128]
    s_ref[1] = sv[:, 128:]
    dinv_ref[...] = dinv

  return pl.pallas_call(
      body,
      grid=(GRID,),
      in_specs=[
          pl.BlockSpec((BN, 128), lambda i: (i, 0)),
          pl.BlockSpec((128, 256), lambda i: (0, 0)),
          pl.BlockSpec((BN, nprt), lambda i: (i, 0)),
      ],
      out_specs=[
          pl.BlockSpec((NSC, BN, 128), lambda i: (0, i, 0)),
          pl.BlockSpec((BN, 1), lambda i: (i, 0)),
      ],
      out_shape=[
          jax.ShapeDtypeStruct((NSC, NR, 128), jnp.float32),
          jax.ShapeDtypeStruct((NR, 1), jnp.float32),
      ],
  )(x, w1, degp)


def _tc_mid(agg, sprev, dinv, b, w, hout, split_out):
  hh_in = agg.shape[2]
  hin2 = 2 * hh_in
  hh_out = hout // 2

  def body(a_ref, s_ref, d_ref, b_ref, w_ref, o_ref):
    af = jnp.concatenate([a_ref[0], a_ref[1]], axis=1)
    sf = jnp.concatenate([s_ref[0], s_ref[1]], axis=1)
    dv = d_ref[...]
    h = jnp.maximum(dv * (af + sf) + b_ref[...], 0.0)
    sv = dv * jnp.dot(h, w_ref[...], preferred_element_type=jnp.float32)
    if split_out:
      o_ref[0] = sv[:, :hh_out]
      o_ref[1] = sv[:, hh_out:]
    else:
      o_ref[...] = sv

  if split_out:
    out_spec = pl.BlockSpec((NSC, BN, hh_out), lambda i: (0, i, 0))
    out_shape = jax.ShapeDtypeStruct((NSC, NR, hh_out), jnp.float32)
  else:
    out_spec = pl.BlockSpec((BN, hout), lambda i: (i, 0))
    out_shape = jax.ShapeDtypeStruct((NR, hout), jnp.float32)

  return pl.pallas_call(
      body,
      grid=(GRID,),
      in_specs=[
          pl.BlockSpec((NSC, BN, hh_in), lambda i: (0, i, 0)),
          pl.BlockSpec((NSC, BN, hh_in), lambda i: (0, i, 0)),
          pl.BlockSpec((BN, 1), lambda i: (i, 0)),
          pl.BlockSpec((1, hin2), lambda i: (0, 0)),
          pl.BlockSpec((hin2, hout), lambda i: (0, 0)),
      ],
      out_specs=out_spec,
      out_shape=out_shape,
  )(agg, sprev, dinv, b, w)


def _tc_final(agg, sprev, dinv, b):
  def body(a_ref, s_ref, d_ref, b_ref, o_ref):
    z = d_ref[...] * (a_ref[0] + a_ref[1] + s_ref[...]) + b_ref[...]
    o_ref[...] = jax.nn.sigmoid(jnp.maximum(z, 0.0))

  return pl.pallas_call(
      body,
      grid=(GRID,),
      in_specs=[
          pl.BlockSpec((NSC, BN, 128), lambda i: (0, i, 0)),
          pl.BlockSpec((BN, 128), lambda i: (i, 0)),
          pl.BlockSpec((BN, 1), lambda i: (i, 0)),
          pl.BlockSpec((1, 128), lambda i: (0, 0)),
      ],
      out_specs=pl.BlockSpec((BN, 128), lambda i: (i, 0)),
      out_shape=jax.ShapeDtypeStruct((NR, 128), jnp.float32),
  )(agg, sprev, dinv, b)


@jax.jit
def kernel(x, edge_index, y, W1, b1, W2, b2, W3, b3):
  del y
  src = edge_index[0].astype(jnp.int32)
  dst = edge_index[1].astype(jnp.int32)
  srcp = jnp.zeros((EPAD16,), jnp.int32).at[:E].set(src)
  srcab = jnp.concatenate([srcp, srcp + NR])
  dstp16 = jnp.full((EPAD16,), N, jnp.int32).at[:E].set(dst)
  srcp32 = jnp.zeros((EPAD32,), jnp.int32).at[:E].set(src)
  dstp32 = jnp.full((EPAD32,), N, jnp.int32).at[:E].set(dst)
  dstd = jnp.full((EPADD,), N, jnp.int32).at[:E].set(dst)
  dstg = dstd.reshape(NW, T32, G, CHUNK)
  ones128 = jnp.ones((CHUNK, 128), jnp.float32)
  z128 = jnp.zeros((TROWS, 128), jnp.float32)
  xp = jnp.zeros((NR, 128), x.dtype).at[:N].set(x)

  hist = _make_deg()(dstg, ones128, z128)
  degp = hist[:, :, 0].T

  agg128 = _make_agg(128)
  s1, dinv = _tc_first(xp, W1, degp)
  agg1 = agg128(s1.reshape(NSC * NR, 128), srcab, dstp16, z128)
  s2 = _tc_mid(agg1, s1, dinv, b1.reshape(1, 256), W2, 256, True)
  agg2 = agg128(s2.reshape(NSC * NR, 128), srcab, dstp16, z128)
  s3 = _tc_mid(agg2, s2, dinv, b2.reshape(1, 256), W3, 128, False)
  agg3 = _make_agg_part()(s3, srcp32, dstp32, z128)
  out = _tc_final(agg3, s3, dinv, b3.reshape(1, 128))
  return out[:N]
